# Initial kernel scaffold; baseline (speedup 1.0000x reference)
import jax
import jax.numpy as jnp
from jax.experimental import pallas as pl


def kernel(x, edge_index, W1, b1, W2, b2):
    raise NotImplementedError("write your pallas kernel here")


# speedup vs baseline: 47.6449x; 47.6449x over previous
import functools

import jax
import jax.numpy as jnp
from jax import lax
from jax.experimental import pallas as pl
from jax.experimental.pallas import tpu as pltpu
from jax.experimental.pallas import tpu_sc as plsc

N = 10000
F_IN = 128
H = 16
C = 128
E = 320000

NC = 2
NS = 16
NW = NC * NS
EPW = E // NW
NP = 10240
ROWS_PER_TILE = NP // NS

BLK = 128
NFULL = EPW // BLK
REM = EPW - NFULL * BLK

_mesh = plsc.VectorSubcoreMesh(core_axis_name="c", subcore_axis_name="s")


@functools.partial(
    pl.kernel,
    out_type=jax.ShapeDtypeStruct((NC, NP), jnp.float32),
    mesh=_mesh,
    scratch_types=[
        pltpu.VMEM((EPW,), jnp.int32),
        pltpu.VMEM((BLK,), jnp.int32),
        pltpu.VMEM((REM,), jnp.int32),
        pltpu.VMEM((BLK,), jnp.float32),
        pltpu.VMEM((REM,), jnp.float32),
        pltpu.VMEM((ROWS_PER_TILE,), jnp.float32),
        pltpu.VMEM_SHARED((NP,), jnp.float32),
    ],
)
def _deg_sc(dst_hbm, degp_hbm, dst_v, iblk, irem, ones_v, ones_r, zeros_v, deg_sh):
    cid = lax.axis_index("c")
    sid = lax.axis_index("s")
    wid = sid * NC + cid

    def _fill(i, _):
        zeros_v[pl.ds(i * 16, 16)] = jnp.zeros((16,), jnp.float32)
        return _
    lax.fori_loop(0, ROWS_PER_TILE // 16, _fill, None)
    for k in range(BLK // 16):
        ones_v[pl.ds(k * 16, 16)] = jnp.ones((16,), jnp.float32)
    ones_r[...] = jnp.ones((REM,), jnp.float32)

    pltpu.sync_copy(zeros_v, deg_sh.at[pl.ds(sid * ROWS_PER_TILE, ROWS_PER_TILE)])
    plsc.subcore_barrier()

    pltpu.sync_copy(dst_hbm.at[pl.ds(wid * EPW, EPW)], dst_v)

    def _blk(j, _):
        for k in range(BLK // 16):
            iblk[pl.ds(k * 16, 16)] = dst_v[pl.ds(j * BLK + k * 16, 16)]
        pltpu.sync_copy(ones_v, deg_sh.at[iblk], add=True)
        return _
    lax.fori_loop(0, NFULL, _blk, None)

    irem[...] = dst_v[pl.ds(NFULL * BLK, REM)]
    pltpu.sync_copy(ones_r, deg_sh.at[irem], add=True)

    plsc.subcore_barrier()
    pltpu.sync_copy(
        deg_sh.at[pl.ds(sid * ROWS_PER_TILE, ROWS_PER_TILE)],
        degp_hbm.at[cid, pl.ds(sid * ROWS_PER_TILE, ROWS_PER_TILE)],
    )


@functools.partial(
    pl.kernel,
    out_type=jax.ShapeDtypeStruct((NC, NP, H), jnp.float32),
    mesh=_mesh,
    scratch_types=[
        pltpu.VMEM((EPW,), jnp.int32),
        pltpu.VMEM((EPW,), jnp.int32),
        pltpu.VMEM((BLK,), jnp.int32),
        pltpu.VMEM((BLK,), jnp.int32),
        pltpu.VMEM((REM,), jnp.int32),
        pltpu.VMEM((REM,), jnp.int32),
        pltpu.VMEM((BLK, H), jnp.float32),
        pltpu.VMEM((REM, H), jnp.float32),
        pltpu.VMEM((ROWS_PER_TILE, H), jnp.float32),
        pltpu.VMEM_SHARED((NP, H), jnp.float32),
        pltpu.SemaphoreType.DMA,
    ],
    compiler_params=pltpu.CompilerParams(use_tc_tiling_on_sc=False),
)
def _agg_sc(g_hbm, src_hbm, dst_hbm, aggp_hbm,
            src_v, dst_v, sblk, dblk, srem, drem, rows, rows_r, zeros_v,
            agg_sh, gsem):
    cid = lax.axis_index("c")
    sid = lax.axis_index("s")
    wid = sid * NC + cid

    def _fill(i, _):
        zeros_v[i] = jnp.zeros((H,), jnp.float32)
        return _
    lax.fori_loop(0, ROWS_PER_TILE, _fill, None)
    pltpu.sync_copy(
        zeros_v,
        agg_sh.at[pl.ds(sid * ROWS_PER_TILE, ROWS_PER_TILE)],
    )
    plsc.subcore_barrier()

    pltpu.sync_copy(src_hbm.at[pl.ds(wid * EPW, EPW)], src_v)
    pltpu.sync_copy(dst_hbm.at[pl.ds(wid * EPW, EPW)], dst_v)

    def _blk(j, _):
        for k in range(BLK // 16):
            sblk[pl.ds(k * 16, 16)] = src_v[pl.ds(j * BLK + k * 16, 16)]
            dblk[pl.ds(k * 16, 16)] = dst_v[pl.ds(j * BLK + k * 16, 16)]
        pltpu.async_copy(g_hbm.at[sblk], rows, gsem).wait()
        pltpu.sync_copy(rows, agg_sh.at[dblk], add=True)
        return _
    lax.fori_loop(0, NFULL, _blk, None)

    srem[...] = src_v[pl.ds(NFULL * BLK, REM)]
    drem[...] = dst_v[pl.ds(NFULL * BLK, REM)]
    pltpu.async_copy(g_hbm.at[srem], rows_r, gsem).wait()
    pltpu.sync_copy(rows_r, agg_sh.at[drem], add=True)

    plsc.subcore_barrier()
    pltpu.sync_copy(
        agg_sh.at[pl.ds(sid * ROWS_PER_TILE, ROWS_PER_TILE)],
        aggp_hbm.at[cid, pl.ds(sid * ROWS_PER_TILE, ROWS_PER_TILE)],
    )


RB = 1280
GRID = NP // RB


def _g_tc(x_ref, w1_ref, degp_ref, g_ref):
    deg = degp_ref[0] + degp_ref[1] + 1.0
    dinv = lax.rsqrt(deg)
    h = jnp.dot(x_ref[...], w1_ref[...], preferred_element_type=jnp.float32)
    g_ref[...] = h * dinv[:, None]


def _g_call(x, W1, degp):
    return pl.pallas_call(
        _g_tc,
        grid=(GRID,),
        in_specs=[
            pl.BlockSpec((RB, F_IN), lambda i: (i, 0)),
            pl.BlockSpec((F_IN, H), lambda i: (0, 0)),
            pl.BlockSpec((NC, RB), lambda i: (0, i)),
        ],
        out_specs=pl.BlockSpec((RB, H), lambda i: (i, 0)),
        out_shape=jax.ShapeDtypeStruct((N, H), jnp.float32),
    )(x, W1, degp)


def _out_tc(aggp_ref, degp_ref, g_ref, b1_ref, w2_ref, b2_ref, o_ref):
    deg = degp_ref[0] + degp_ref[1] + 1.0
    dinv = lax.rsqrt(deg)
    a = (aggp_ref[0] + aggp_ref[1] + g_ref[...]) * dinv[:, None] + b1_ref[...]
    r = jnp.maximum(a, 0.0)
    o_ref[...] = jnp.dot(r, w2_ref[...], preferred_element_type=jnp.float32) + b2_ref[...]


def _out_call(aggp, degp, g, b1, W2, b2):
    return pl.pallas_call(
        _out_tc,
        grid=(GRID,),
        in_specs=[
            pl.BlockSpec((NC, RB, H), lambda i: (0, i, 0)),
            pl.BlockSpec((NC, RB), lambda i: (0, i)),
            pl.BlockSpec((RB, H), lambda i: (i, 0)),
            pl.BlockSpec((1, H), lambda i: (0, 0)),
            pl.BlockSpec((H, C), lambda i: (0, 0)),
            pl.BlockSpec((1, C), lambda i: (0, 0)),
        ],
        out_specs=pl.BlockSpec((RB, C), lambda i: (i, 0)),
        out_shape=jax.ShapeDtypeStruct((N, C), jnp.float32),
    )(aggp, degp, g, b1, W2, b2)


def kernel(x, edge_index, W1, b1, W2, b2):
    src = edge_index[0]
    dst = edge_index[1]
    degp = _deg_sc(dst)
    g = _g_call(x, W1, degp)
    aggp = _agg_sc(g, src, dst)
    out = _out_call(aggp, degp, g, b1.reshape(1, H), W2, b2.reshape(1, C))
    return out
